# baseline (speedup 1.0000x reference)
import functools

import jax
import jax.numpy as jnp
from jax import lax
from jax.experimental import pallas as pl
from jax.experimental.pallas import tpu as pltpu
from jax.experimental.pallas import tpu_sc as plsc

B, T, S = 64, 4096, 16
BB = 16
BIG = 65536.0
N_ELEMS = float(B * T * S)

_INFO = plsc.get_sparse_core_info()
NC, NS = _INFO.num_cores, _INFO.num_subcores
NW = NC * NS
BPW = B // NW


def _bce_main(pred_ref, tgt_ref, m_ref, k_ref, neg_ref, acc_ref):
    g = pl.program_id(0)

    @pl.when(g == 0)
    def _():
        acc_ref[0, 0] = 0.0

    tval = lax.broadcasted_iota(jnp.int32, (S, T), 1).astype(jnp.float32)
    i_col = lax.broadcasted_iota(jnp.int32, (S, 1), 0).astype(jnp.float32)
    eye = (lax.broadcasted_iota(jnp.int32, (S, S), 0) ==
           lax.broadcasted_iota(jnp.int32, (S, S), 1)).astype(jnp.float32)

    total = jnp.zeros((), jnp.float32)
    for bb in range(BB):
        p = pred_ref[bb]
        t = tgt_ref[bb]

        lp = jnp.maximum(jnp.log(p), -100.0)
        l1p = jnp.maximum(jnp.log(1.0 - p), -100.0)
        d = lp - l1p

        m16 = lax.dot_general(t, d, (((1,), (1,)), ((), ())),
                              preferred_element_type=jnp.float32)
        m_ref[bb] = m16

        cand = jnp.where(t > 0.0, tval, BIG)
        o_col = jnp.min(cand, axis=1, keepdims=True)

        k_col = o_col * 16.0 + i_col
        kcol = jnp.broadcast_to(k_col, (S, S))
        krow = lax.dot_general(kcol, eye, (((0,), (0,)), ((), ())),
                               preferred_element_type=jnp.float32)
        k_ref[bb:bb + 1, :] = krow[0:1, :]

        total = total - jnp.sum(l1p)

    acc_ref[0, 0] = acc_ref[0, 0] + total

    @pl.when(g == B // BB - 1)
    def _():
        neg_ref[...] = jnp.reshape(acc_ref[0, 0], (1, 1))


@functools.partial(
    pl.kernel,
    mesh=plsc.VectorSubcoreMesh(core_axis_name="c", subcore_axis_name="s"),
    out_type=jax.ShapeDtypeStruct((NW, S), jnp.float32),
    compiler_params=pltpu.CompilerParams(needs_layout_passes=False),
    scratch_types=[
        pltpu.VMEM((S,), jnp.float32),
        pltpu.VMEM((S, S), jnp.float32),
        pltpu.VMEM((S,), jnp.float32),
    ],
)
def _sc_sort_select(m_hbm, k_hbm, out_hbm, kv_v, rows_v, acc_v):
    wid = lax.axis_index("s") * NC + lax.axis_index("c")
    iota = lax.iota(jnp.int32, S)
    acc = jnp.zeros((S,), jnp.float32)
    for k in range(BPW):
        b = wid * BPW + k
        pltpu.sync_copy(k_hbm.at[b], kv_v)
        _, sigma = plsc.sort_key_val(kv_v[...], iota)
        pltpu.sync_copy(m_hbm.at[pl.ds(b * S, S)], rows_v)
        acc = acc + plsc.load_gather(rows_v, [sigma, iota])
    acc_v[...] = acc
    pltpu.sync_copy(acc_v, out_hbm.at[wid])


@jax.jit
def kernel(predictions, targets):
    pr = jnp.transpose(predictions, (0, 2, 1))
    tg = jnp.transpose(targets, (0, 2, 1))
    spec = pl.BlockSpec((BB, S, T), lambda b: (b, 0, 0))
    m_out, k_out, neg = pl.pallas_call(
        _bce_main,
        grid=(B // BB,),
        in_specs=[spec, spec],
        out_specs=[
            pl.BlockSpec((BB, S, S), lambda b: (b, 0, 0)),
            pl.BlockSpec((BB, S), lambda b: (b, 0)),
            pl.BlockSpec((1, 1), lambda b: (0, 0)),
        ],
        out_shape=[
            jax.ShapeDtypeStruct((B, S, S), jnp.float32),
            jax.ShapeDtypeStruct((B, S), jnp.float32),
            jax.ShapeDtypeStruct((1, 1), jnp.float32),
        ],
        scratch_shapes=[pltpu.SMEM((1, 1), jnp.float32)],
    )(pr, tg)
    cross_parts = _sc_sort_select(m_out.reshape(B * S, S), k_out)
    return ((neg[0, 0] - jnp.sum(cross_parts)) * (1.0 / N_ELEMS))


# speedup vs baseline: 1.0966x; 1.0966x over previous
import functools

import jax
import jax.numpy as jnp
from jax import lax
from jax.experimental import pallas as pl
from jax.experimental.pallas import tpu as pltpu
from jax.experimental.pallas import tpu_sc as plsc

B, T, S = 64, 4096, 16
BB = 16
BIG = 65536.0
N_ELEMS = float(B * T * S)

_INFO = plsc.get_sparse_core_info()
NC, NS = _INFO.num_cores, _INFO.num_subcores
NW = NC * NS
BPW = B // NW


def _bce_main(pred_ref, tgt_ref, m_ref, k_ref, neg_ref, acc_ref):
    g = pl.program_id(0)

    @pl.when(g == 0)
    def _():
        acc_ref[0, 0] = 0.0

    tval = lax.broadcasted_iota(jnp.int32, (S, T), 1).astype(jnp.float32)
    i_col = lax.broadcasted_iota(jnp.int32, (S, 1), 0).astype(jnp.float32)
    eye = (lax.broadcasted_iota(jnp.int32, (S, S), 0) ==
           lax.broadcasted_iota(jnp.int32, (S, S), 1)).astype(jnp.float32)

    total = jnp.zeros((), jnp.float32)
    for bb in range(BB):
        p = pred_ref[bb]
        t = tgt_ref[bb]

        lp = jnp.maximum(jnp.log(p), -100.0)
        l1p = jnp.maximum(jnp.log(1.0 - p), -100.0)
        d = lp - l1p

        m16 = lax.dot_general(t, d, (((1,), (1,)), ((), ())),
                              preferred_element_type=jnp.float32)
        m_ref[bb] = m16

        cand = jnp.where(t > 0.0, tval, BIG)
        o_col = jnp.min(cand, axis=1, keepdims=True)

        k_col = o_col * 16.0 + i_col
        kcol = jnp.broadcast_to(k_col, (S, S))
        krow = lax.dot_general(kcol, eye, (((0,), (0,)), ((), ())),
                               preferred_element_type=jnp.float32)
        k_ref[bb:bb + 1, :] = krow[0:1, :]

        total = total - jnp.sum(l1p)

    acc_ref[0, 0] = acc_ref[0, 0] + total

    @pl.when(g == B // BB - 1)
    def _():
        neg_ref[...] = jnp.broadcast_to(
            jnp.reshape(acc_ref[0, 0], (1, 1)), (1, S))


BPS = B // NS


@functools.partial(
    pl.kernel,
    mesh=plsc.VectorSubcoreMesh(core_axis_name="c", subcore_axis_name="s"),
    out_type=jax.ShapeDtypeStruct((S,), jnp.float32),
    compiler_params=pltpu.CompilerParams(needs_layout_passes=False),
    scratch_types=[
        pltpu.VMEM((BPS, S), jnp.float32),
        pltpu.VMEM((BPS * S, S), jnp.float32),
        pltpu.VMEM((S,), jnp.float32),
        pltpu.VMEM((S,), jnp.float32),
        pltpu.VMEM_SHARED((S,), jnp.float32),
    ],
)
def _sc_sort_select(m_hbm, k_hbm, neg_hbm, out_hbm,
                    keys_v, rows_v, acc_v, neg_v, shared):
    cid = lax.axis_index("c")
    sid = lax.axis_index("s")

    @pl.when(cid == 0)
    def _():
        iota = lax.iota(jnp.int32, S)

        @pl.when(sid == 0)
        def _():
            acc_v[...] = jnp.zeros((S,), jnp.float32)
            pltpu.sync_copy(acc_v, shared)
        plsc.subcore_barrier()

        b0 = sid * BPS
        pltpu.sync_copy(k_hbm.at[pl.ds(b0, BPS)], keys_v)
        pltpu.sync_copy(m_hbm.at[pl.ds(b0 * S, BPS * S)], rows_v)
        acc = jnp.zeros((S,), jnp.float32)
        for k in range(BPS):
            _, sigma = plsc.sort_key_val(keys_v[k], iota)
            acc = acc + plsc.load_gather(rows_v, [k * S + sigma, iota])
        acc_v[...] = acc
        pltpu.sync_copy(acc_v, shared.at[lax.iota(jnp.int32, S)], add=True)
        plsc.subcore_barrier()

        @pl.when(sid == 0)
        def _():
            pltpu.sync_copy(shared, acc_v)
            pltpu.sync_copy(neg_hbm.at[0], neg_v)
            cross = jnp.sum(acc_v[...])
            neg_s = jnp.sum(neg_v[...]) * (1.0 / S)
            res = (neg_s - cross) * (1.0 / N_ELEMS)
            acc_v[...] = jnp.full((S,), res, jnp.float32)
            pltpu.sync_copy(acc_v, out_hbm)


@jax.jit
def kernel(predictions, targets):
    pr = jnp.transpose(predictions, (0, 2, 1))
    tg = jnp.transpose(targets, (0, 2, 1))
    spec = pl.BlockSpec((BB, S, T), lambda b: (b, 0, 0))
    m_out, k_out, neg = pl.pallas_call(
        _bce_main,
        grid=(B // BB,),
        in_specs=[spec, spec],
        out_specs=[
            pl.BlockSpec((BB, S, S), lambda b: (b, 0, 0)),
            pl.BlockSpec((BB, S), lambda b: (b, 0)),
            pl.BlockSpec((1, S), lambda b: (0, 0)),
        ],
        out_shape=[
            jax.ShapeDtypeStruct((B, S, S), jnp.float32),
            jax.ShapeDtypeStruct((B, S), jnp.float32),
            jax.ShapeDtypeStruct((1, S), jnp.float32),
        ],
        scratch_shapes=[pltpu.SMEM((1, 1), jnp.float32)],
    )(pr, tg)
    res = _sc_sort_select(m_out.reshape(B * S, S), k_out, neg)
    return res[0]
